# baseline (speedup 1.0000x reference)
import jax
import jax.numpy as jnp
from jax import lax
from jax.experimental import pallas as pl
from jax.experimental.pallas import tpu as pltpu
from jax.experimental.pallas import tpu_sc as plsc

VOCAB = 100000
EMBED = 64
BATCH = 1024
CTX = 20

NUM_CORES = 2
NUM_SUBCORES = 16
NUM_WORKERS = NUM_CORES * NUM_SUBCORES
DIMS_PER_W = EMBED // NUM_WORKERS

LANES = 16
N_GROUPS = BATCH // LANES


CHUNK0 = 50048
CHUNK1_OFF = 49152
CHUNK1 = VOCAB - CHUNK1_OFF
CHUNKS = ((0, CHUNK0), (CHUNK1_OFF, CHUNK1))
OVERLAP = CHUNK0 - CHUNK1_OFF
BUF_LEN = max(CHUNK0, CHUNK1)


def _pool_t_body(idxt_hbm, et_hbm, pooledt_hbm, idx_v, buf_a, buf_b,
                 pooled_v, sem_a, sem_b, sem_i):
    wid = lax.axis_index("s") * NUM_CORES + lax.axis_index("c")

    idx_desc = pltpu.async_copy(idxt_hbm, idx_v, sem_i)

    inv_ctx = jnp.float32(1.0 / CTX)
    bufs = (buf_a, buf_b)
    sems = (sem_a, sem_b)
    passes = [(dl, h) for dl in range(DIMS_PER_W) for h in range(2)]

    def start(p):
        dl, h = passes[p]
        off, ln = CHUNKS[h]
        d = wid * DIMS_PER_W + dl
        return pltpu.async_copy(
            et_hbm.at[d, pl.ds(off, ln)],
            bufs[p % 2].at[pl.ds(0, ln)],
            sems[p % 2],
        )

    descs = [None] * len(passes)
    descs[0] = start(0)
    idx_desc.wait()

    for p, (dl, h) in enumerate(passes):
        if p + 1 < len(passes):
            descs[p + 1] = start(p + 1)
        descs[p].wait()
        buf = bufs[p % 2]
        off, ln = CHUNKS[h]

        def grp(g, carry):
            base = g * LANES
            acc = jnp.zeros((LANES,), jnp.float32)
            for c in range(CTX):
                iv = idx_v[c, pl.ds(base, LANES)]
                if h == 0:
                    m = iv < ln
                    lc = jnp.minimum(iv, ln - 1)
                else:
                    lc0 = iv - off
                    m = lc0 >= OVERLAP
                    lc = jnp.maximum(lc0, 0)
                g16 = plsc.load_gather(buf, [lc])
                acc = acc + jnp.where(m, g16, jnp.float32(0.0))
            if h == 0:
                pooled_v[pl.ds(base, LANES)] = acc
            else:
                pooled_v[pl.ds(base, LANES)] = (
                    pooled_v[pl.ds(base, LANES)] + acc
                ) * inv_ctx
            return carry

        lax.fori_loop(0, N_GROUPS, grp, 0)

        if h == 1:
            d = wid * DIMS_PER_W + dl
            pltpu.sync_copy(pooled_v, pooledt_hbm.at[d])


def _pool_t(idx_t, e_t):
    pool = pl.kernel(
        _pool_t_body,
        out_type=jax.ShapeDtypeStruct((EMBED, BATCH), jnp.float32),
        mesh=plsc.VectorSubcoreMesh(core_axis_name="c", subcore_axis_name="s"),
        scratch_types=[
            pltpu.VMEM((CTX, BATCH), jnp.int32),
            pltpu.VMEM((BUF_LEN,), jnp.float32),
            pltpu.VMEM((BUF_LEN,), jnp.float32),
            pltpu.VMEM((BATCH,), jnp.float32),
            pltpu.SemaphoreType.DMA,
            pltpu.SemaphoreType.DMA,
            pltpu.SemaphoreType.DMA,
        ],
        compiler_params=pltpu.CompilerParams(needs_layout_passes=False),
    )
    return pool(idx_t, e_t)


def _matmul_body(pooledt_ref, w_ref, b_ref, out_ref):
    acc = lax.dot_general(
        w_ref[...], pooledt_ref[...],
        (((0,), (0,)), ((), ())),
        preferred_element_type=jnp.float32,
    )
    ones = jnp.ones((1, BATCH), jnp.float32)
    bias = lax.dot_general(
        b_ref[...], ones,
        (((0,), (0,)), ((), ())),
        preferred_element_type=jnp.float32,
    )
    out_ref[...] = acc + bias


BN = 2048


def _project_t(pooled_t, w, b2):
    grid = (pl.cdiv(VOCAB, BN),)
    return pl.pallas_call(
        _matmul_body,
        grid=grid,
        in_specs=[
            pl.BlockSpec((EMBED, BATCH), lambda j: (0, 0)),
            pl.BlockSpec((EMBED, BN), lambda j: (0, j)),
            pl.BlockSpec((1, BN), lambda j: (0, j)),
        ],
        out_specs=pl.BlockSpec((BN, BATCH), lambda j: (j, 0)),
        out_shape=jax.ShapeDtypeStruct((VOCAB, BATCH), jnp.float32),
        compiler_params=pltpu.CompilerParams(
            dimension_semantics=("arbitrary",),
        ),
    )(pooled_t, w, b2)


@jax.jit
def kernel(inputs, E, W, b):
    idx_t = inputs.astype(jnp.int32).T
    e_t = E.T
    pooled_t = _pool_t(idx_t, e_t)
    return _project_t(pooled_t, W, b.reshape(1, VOCAB)).T


# speedup vs baseline: 1.0248x; 1.0248x over previous
import jax
import jax.numpy as jnp
from jax import lax
from jax.experimental import pallas as pl
from jax.experimental.pallas import tpu as pltpu
from jax.experimental.pallas import tpu_sc as plsc

VOCAB = 100000
EMBED = 64
BATCH = 1024
CTX = 20

NUM_CORES = 2
NUM_SUBCORES = 16
NUM_WORKERS = NUM_CORES * NUM_SUBCORES
DIMS_PER_W = EMBED // NUM_WORKERS

LANES = 16
N_GROUPS = BATCH // LANES


def _pool_t_body(idxt_hbm, et_hbm, pooledt_hbm, idx_v, row_v, pooled_v,
                 sem, sem_i):
    wid = lax.axis_index("s") * NUM_CORES + lax.axis_index("c")

    row_desc = pltpu.async_copy(et_hbm.at[wid * DIMS_PER_W], row_v, sem)
    idx_desc = pltpu.async_copy(idxt_hbm, idx_v, sem_i)
    idx_desc.wait()

    inv_ctx = jnp.float32(1.0 / CTX)

    for d_local in range(DIMS_PER_W):
        d = wid * DIMS_PER_W + d_local
        if d_local > 0:
            row_desc = pltpu.async_copy(et_hbm.at[d], row_v, sem)
        row_desc.wait()

        def grp(g, carry):
            base = g * LANES
            iv = idx_v[0, pl.ds(base, LANES)]
            acc = plsc.load_gather(row_v, [iv])
            for c in range(1, CTX):
                iv = idx_v[c, pl.ds(base, LANES)]
                acc = acc + plsc.load_gather(row_v, [iv])
            pooled_v[pl.ds(base, LANES)] = acc * inv_ctx
            return carry

        lax.fori_loop(0, N_GROUPS, grp, 0)

        pltpu.sync_copy(pooled_v, pooledt_hbm.at[d])


def _pool_t(idx_t, e_t):
    pool = pl.kernel(
        _pool_t_body,
        out_type=jax.ShapeDtypeStruct((EMBED, BATCH), jnp.float32),
        mesh=plsc.VectorSubcoreMesh(core_axis_name="c", subcore_axis_name="s"),
        scratch_types=[
            pltpu.VMEM((CTX, BATCH), jnp.int32),
            pltpu.VMEM((VOCAB,), jnp.float32),
            pltpu.VMEM((BATCH,), jnp.float32),
            pltpu.SemaphoreType.DMA,
            pltpu.SemaphoreType.DMA,
        ],
        compiler_params=pltpu.CompilerParams(needs_layout_passes=False),
    )
    return pool(idx_t, e_t)


def _matmul_body(pooledt_ref, w_ref, b_ref, out_ref):
    acc = lax.dot_general(
        w_ref[...], pooledt_ref[...],
        (((0,), (0,)), ((), ())),
        preferred_element_type=jnp.float32,
    )
    ones = jnp.ones((1, BATCH), jnp.float32)
    bias = lax.dot_general(
        b_ref[...], ones,
        (((0,), (0,)), ((), ())),
        preferred_element_type=jnp.float32,
    )
    out_ref[...] = acc + bias


BN = 2048


def _project_t(pooled_t, w, b2):
    grid = (pl.cdiv(VOCAB, BN),)
    return pl.pallas_call(
        _matmul_body,
        grid=grid,
        in_specs=[
            pl.BlockSpec((EMBED, BATCH), lambda j: (0, 0)),
            pl.BlockSpec((EMBED, BN), lambda j: (0, j)),
            pl.BlockSpec((1, BN), lambda j: (0, j)),
        ],
        out_specs=pl.BlockSpec((BN, BATCH), lambda j: (j, 0)),
        out_shape=jax.ShapeDtypeStruct((VOCAB, BATCH), jnp.float32),
        compiler_params=pltpu.CompilerParams(
            dimension_semantics=("arbitrary",),
        ),
    )(pooled_t, w, b2)


@jax.jit
def kernel(inputs, E, W, b):
    idx_t = inputs.astype(jnp.int32).T
    e_t = E.T
    pooled_t = _pool_t(idx_t, e_t)
    return _project_t(pooled_t, W, b.reshape(1, VOCAB)).T


# speedup vs baseline: 1.0326x; 1.0076x over previous
import jax
import jax.numpy as jnp
from jax import lax
from jax.experimental import pallas as pl
from jax.experimental.pallas import tpu as pltpu
from jax.experimental.pallas import tpu_sc as plsc

VOCAB = 100000
EMBED = 64
BATCH = 1024
CTX = 20

NUM_CORES = 2
NUM_SUBCORES = 16
NUM_WORKERS = NUM_CORES * NUM_SUBCORES
DIMS_PER_W = EMBED // NUM_WORKERS

LANES = 16
N_GROUPS = BATCH // LANES


def _pool_t_body(idxt_hbm, et_hbm, pooledt_hbm, idx_v, row_v, pooled_v,
                 sem, sem_i):
    wid = lax.axis_index("s") * NUM_CORES + lax.axis_index("c")

    row_desc = pltpu.async_copy(et_hbm.at[wid * DIMS_PER_W], row_v, sem)
    idx_desc = pltpu.async_copy(idxt_hbm, idx_v, sem_i)
    idx_desc.wait()

    inv_ctx = jnp.float32(1.0 / CTX)

    for d_local in range(DIMS_PER_W):
        d = wid * DIMS_PER_W + d_local
        if d_local > 0:
            row_desc = pltpu.async_copy(et_hbm.at[d], row_v, sem)
        row_desc.wait()

        def grp(g, carry):
            base = g * LANES
            iv = idx_v[0, pl.ds(base, LANES)]
            acc = plsc.load_gather(row_v, [iv])
            for c in range(1, CTX):
                iv = idx_v[c, pl.ds(base, LANES)]
                acc = acc + plsc.load_gather(row_v, [iv])
            pooled_v[pl.ds(base, LANES)] = acc * inv_ctx
            return carry

        lax.fori_loop(0, N_GROUPS, grp, 0)

        pltpu.sync_copy(pooled_v, pooledt_hbm.at[d])


def _pool_t(idx_t, e_t):
    pool = pl.kernel(
        _pool_t_body,
        out_type=jax.ShapeDtypeStruct((EMBED, BATCH), jnp.float32),
        mesh=plsc.VectorSubcoreMesh(core_axis_name="c", subcore_axis_name="s"),
        scratch_types=[
            pltpu.VMEM((CTX, BATCH), jnp.int32),
            pltpu.VMEM((VOCAB,), jnp.float32),
            pltpu.VMEM((BATCH,), jnp.float32),
            pltpu.SemaphoreType.DMA,
            pltpu.SemaphoreType.DMA,
        ],
        compiler_params=pltpu.CompilerParams(needs_layout_passes=False),
    )
    return pool(idx_t, e_t)


def _matmul_body(pooledt_ref, w_ref, b_ref, out_ref):
    acc = lax.dot_general(
        w_ref[...], pooledt_ref[...],
        (((0,), (0,)), ((), ())),
        preferred_element_type=jnp.float32,
    )
    ones = jnp.ones((1, BATCH), jnp.float32)
    bias = lax.dot_general(
        b_ref[...], ones,
        (((0,), (0,)), ((), ())),
        preferred_element_type=jnp.float32,
    )
    out_ref[...] = acc + bias


BN = 4096


def _project_t(pooled_t, w, b2):
    grid = (pl.cdiv(VOCAB, BN),)
    return pl.pallas_call(
        _matmul_body,
        grid=grid,
        in_specs=[
            pl.BlockSpec((EMBED, BATCH), lambda j: (0, 0)),
            pl.BlockSpec((EMBED, BN), lambda j: (0, j)),
            pl.BlockSpec((1, BN), lambda j: (0, j)),
        ],
        out_specs=pl.BlockSpec((BN, BATCH), lambda j: (j, 0)),
        out_shape=jax.ShapeDtypeStruct((VOCAB, BATCH), jnp.float32),
        compiler_params=pltpu.CompilerParams(
            dimension_semantics=("arbitrary",),
        ),
    )(pooled_t, w, b2)


@jax.jit
def kernel(inputs, E, W, b):
    idx_t = inputs.astype(jnp.int32).T
    e_t = E.T
    pooled_t = _pool_t(idx_t, e_t)
    return _project_t(pooled_t, W, b.reshape(1, VOCAB)).T
